# baseline (speedup 1.0000x reference)
import jax
import jax.numpy as jnp
from jax import lax
from jax.experimental import pallas as pl
from jax.experimental.pallas import tpu as pltpu
from jax.experimental.pallas import tpu_sc as plsc

H = 512
HI = 510
ZROW = H * H
NW = 32
ROWS_PER_W = H // NW
NTAB = H * H + 1032


def _sc_warp(table48, imap_pad):
    mesh = plsc.VectorSubcoreMesh(core_axis_name="c", subcore_axis_name="s")

    def body(tab_hbm, imap_hbm, out_hbm, mb_v, idx_v, g_v, acc_v, gsem):
        wid = lax.axis_index("s") * 2 + lax.axis_index("c")
        y0 = wid * ROWS_PER_W

        pltpu.sync_copy(imap_hbm.at[pl.ds(y0, 24)], mb_v)

        zrow = jnp.full((16,), ZROW, jnp.int32)
        zero = jnp.zeros((16,), jnp.int32)
        hi = jnp.full((16,), HI, jnp.int32)
        fzero = jnp.zeros((16,), jnp.float32)

        def dec_row(r, _):
            def dec_col(c, _):
                v = mb_v[r, pl.ds(c * 16, 16)]
                q = lax.div(v, hi)
                mb_v[r, pl.ds(c * 16, 16)] = jnp.where(v < zero, zrow, v + q + q)
                return 0
            return lax.fori_loop(0, 33, dec_col, 0)
        lax.fori_loop(0, 18, dec_row, 0)

        for buf in range(2):
            for r in range(512, 516):
                for w in range(3):
                    g_v[buf, r, pl.ds(w * 16, 16)] = fzero

        def build_and_fire(y, ki, buf):
            row = y + 2 - ki
            offv = jnp.full((16,), ki * H, jnp.int32)

            def bld(c, _):
                vb = mb_v[row, pl.ds(c * 16, 16)]
                q = lax.shift_right_logical(c, 3)
                idx_v[buf, q, pl.ds((c & 7) * 16, 16)] = vb + offv
                return 0
            lax.fori_loop(0, 32, bld, 0)
            for q in range(4):
                pltpu.async_copy(
                    tab_hbm.at[idx_v.at[buf, q]],
                    g_v.at[buf, pl.ds(q * 128, 128)], gsem)

        def do_stage(s, y, ki, buf):
            last = ki == 2
            y1 = jnp.where(last, y + 1, y)
            k1 = jnp.where(last, 0, ki + 1)

            @pl.when(s < 47)
            def _():
                build_and_fire(y1, k1, 1 - buf)

            for q in range(4):
                pltpu.make_async_copy(
                    tab_hbm.at[idx_v.at[buf, q]],
                    g_v.at[buf, pl.ds(q * 128, 128)], gsem).wait()

            def tap3(p):
                s0 = g_v[buf, p + 2, pl.ds(0, 16)]
                s1 = g_v[buf, p + 1, pl.ds(16, 16)]
                s2 = g_v[buf, p, pl.ds(32, 16)]
                return (s0 + s1) + s2

            @pl.when(ki == 0)
            def _():
                def sum_store(p, _):
                    acc_v[p, :] = tap3(p)
                    return 0
                lax.fori_loop(0, H, sum_store, 0)

            @pl.when(ki != 0)
            def _():
                def sum_add(p, _):
                    plsc.addupdate(acc_v.at[p], tap3(p))
                    return 0
                lax.fori_loop(0, H, sum_add, 0)

            @pl.when(last)
            def _():
                pltpu.sync_copy(acc_v, out_hbm.at[pl.ds((y0 + y) * H, H)])
            return y1, k1

        build_and_fire(jnp.int32(0), jnp.int32(0), 0)

        def pair(it, carry):
            y, ki = carry
            s = it * 2
            y, ki = do_stage(s, y, ki, 0)
            y, ki = do_stage(s + 1, y, ki, 1)
            return y, ki
        lax.fori_loop(0, 24, pair, (jnp.int32(0), jnp.int32(0)))

    fn = pl.kernel(
        body,
        out_type=jax.ShapeDtypeStruct((H * H, 16), jnp.float32),
        mesh=mesh,
        scratch_types=[
            pltpu.VMEM((24, 528), jnp.int32),
            pltpu.VMEM((2, 4, 128), jnp.int32),
            pltpu.VMEM((2, 516, 48), jnp.float32),
            pltpu.VMEM((H, 16), jnp.float32),
            pltpu.SemaphoreType.DMA,
        ],
        compiler_params=pltpu.CompilerParams(use_tc_tiling_on_sc=False),
    )
    return fn(table48, imap_pad)


@jax.jit
def kernel(lr, ref, index_map):
    del lr
    padded = jnp.pad(ref, ((0, 0), (0, 0), (1, 1), (1, 1)), mode='reflect')
    flat = padded[0].transpose(1, 2, 0).reshape(-1)
    flatz = jnp.concatenate(
        [flat, jnp.zeros(16 * NTAB + 32 - flat.shape[0], flat.dtype)])
    table48 = jnp.concatenate(
        [flatz[0:16 * NTAB].reshape(NTAB, 16),
         flatz[16:16 * NTAB + 16].reshape(NTAB, 16),
         flatz[32:16 * NTAB + 32].reshape(NTAB, 16)], axis=1)
    m2 = index_map.reshape(HI, HI).astype(jnp.int32)
    imp = jnp.full((520, 528), -1, jnp.int32)
    imp = lax.dynamic_update_slice(imp, m2, (2, 2))
    out = _sc_warp(table48, imp)
    return out.reshape(H, H, 16).transpose(2, 0, 1)[None]


# speedup vs baseline: 1.0093x; 1.0093x over previous
import jax
import jax.numpy as jnp
from jax import lax
from jax.experimental import pallas as pl
from jax.experimental.pallas import tpu as pltpu
from jax.experimental.pallas import tpu_sc as plsc

H = 512
HI = 510
ZROW = H * H
NW = 32
ROWS_PER_W = H // NW
NTAB = H * H + 1032


def _sc_warp(table48, imap_pad):
    mesh = plsc.VectorSubcoreMesh(core_axis_name="c", subcore_axis_name="s")

    def body(tab_hbm, imap_hbm, out_hbm, mb_v, idx_v, g_v, acc_v,
             gs0, gs1, gs2, gs3):
        gsems = (gs0, gs1, gs2, gs3)
        wid = lax.axis_index("s") * 2 + lax.axis_index("c")
        y0 = wid * ROWS_PER_W

        pltpu.sync_copy(imap_hbm.at[pl.ds(y0, 24)], mb_v)

        zrow = jnp.full((16,), ZROW, jnp.int32)
        zero = jnp.zeros((16,), jnp.int32)
        hi = jnp.full((16,), HI, jnp.int32)
        fzero = jnp.zeros((16,), jnp.float32)

        def dec_row(r, _):
            def dec_col(c, _):
                v = mb_v[r, pl.ds(c * 16, 16)]
                q = lax.div(v, hi)
                mb_v[r, pl.ds(c * 16, 16)] = jnp.where(v < zero, zrow, v + q + q)
                return 0
            return lax.fori_loop(0, 33, dec_col, 0)
        lax.fori_loop(0, 18, dec_row, 0)

        for buf in range(4):
            for r in range(512, 516):
                for w in range(3):
                    g_v[buf, r, pl.ds(w * 16, 16)] = fzero

        def build_and_fire(y, ki, buf):
            row = y + 2 - ki
            offv = jnp.full((16,), ki * H, jnp.int32)

            def bld(c, _):
                vb = mb_v[row, pl.ds(c * 16, 16)]
                q = lax.shift_right_logical(c, 3)
                idx_v[buf, q, pl.ds((c & 7) * 16, 16)] = vb + offv
                return 0
            lax.fori_loop(0, 32, bld, 0)
            for q in range(4):
                pltpu.async_copy(
                    tab_hbm.at[idx_v.at[buf, q]],
                    g_v.at[buf, pl.ds(q * 128, 128)], gsems[buf])

        def fire_ahead(s3, y, ki, fbuf):
            @pl.when(s3 < 48)
            def _():
                build_and_fire(y + 1, ki, fbuf)

        def do_stage(s, y, ki, buf):
            last = ki == 2
            y1 = jnp.where(last, y + 1, y)
            k1 = jnp.where(last, 0, ki + 1)

            fire_ahead(s + 3, y, ki, (buf + 3) % 4)

            for q in range(4):
                pltpu.make_async_copy(
                    tab_hbm.at[idx_v.at[buf, q]],
                    g_v.at[buf, pl.ds(q * 128, 128)], gsems[buf]).wait()

            def tap3(p):
                s0 = g_v[buf, p + 2, pl.ds(0, 16)]
                s1 = g_v[buf, p + 1, pl.ds(16, 16)]
                s2 = g_v[buf, p, pl.ds(32, 16)]
                return (s0 + s1) + s2

            @pl.when(ki == 0)
            def _():
                def sum_store(p, _):
                    acc_v[p, :] = tap3(p)
                    return 0
                lax.fori_loop(0, H, sum_store, 0, unroll=4)

            @pl.when(ki != 0)
            def _():
                def sum_add(p, _):
                    plsc.addupdate(acc_v.at[p], tap3(p))
                    return 0
                lax.fori_loop(0, H, sum_add, 0, unroll=4)

            @pl.when(last)
            def _():
                pltpu.sync_copy(acc_v, out_hbm.at[pl.ds((y0 + y) * H, H)])
            return y1, k1

        for s in range(3):
            build_and_fire(jnp.int32(0), jnp.int32(s), s)

        def quad(it, carry):
            y, ki = carry
            s = it * 4
            y, ki = do_stage(s, y, ki, 0)
            y, ki = do_stage(s + 1, y, ki, 1)
            y, ki = do_stage(s + 2, y, ki, 2)
            y, ki = do_stage(s + 3, y, ki, 3)
            return y, ki
        lax.fori_loop(0, 12, quad, (jnp.int32(0), jnp.int32(0)))

    fn = pl.kernel(
        body,
        out_type=jax.ShapeDtypeStruct((H * H, 16), jnp.float32),
        mesh=mesh,
        scratch_types=[
            pltpu.VMEM((24, 528), jnp.int32),
            pltpu.VMEM((4, 4, 128), jnp.int32),
            pltpu.VMEM((4, 516, 48), jnp.float32),
            pltpu.VMEM((H, 16), jnp.float32),
            pltpu.SemaphoreType.DMA,
            pltpu.SemaphoreType.DMA,
            pltpu.SemaphoreType.DMA,
            pltpu.SemaphoreType.DMA,
        ],
        compiler_params=pltpu.CompilerParams(use_tc_tiling_on_sc=False),
    )
    return fn(table48, imap_pad)


@jax.jit
def kernel(lr, ref, index_map):
    del lr
    padded = jnp.pad(ref, ((0, 0), (0, 0), (1, 1), (1, 1)), mode='reflect')
    flat = padded[0].transpose(1, 2, 0).reshape(-1)
    flatz = jnp.concatenate(
        [flat, jnp.zeros(16 * NTAB + 32 - flat.shape[0], flat.dtype)])
    table48 = jnp.concatenate(
        [flatz[0:16 * NTAB].reshape(NTAB, 16),
         flatz[16:16 * NTAB + 16].reshape(NTAB, 16),
         flatz[32:16 * NTAB + 32].reshape(NTAB, 16)], axis=1)
    m2 = index_map.reshape(HI, HI).astype(jnp.int32)
    imp = jnp.full((520, 528), -1, jnp.int32)
    imp = lax.dynamic_update_slice(imp, m2, (2, 2))
    out = _sc_warp(table48, imp)
    return out.reshape(H, H, 16).transpose(2, 0, 1)[None]


# speedup vs baseline: 1.9754x; 1.9571x over previous
import jax
import jax.numpy as jnp
from jax import lax
from jax.experimental import pallas as pl
from jax.experimental.pallas import tpu as pltpu
from jax.experimental.pallas import tpu_sc as plsc

H = 512
HI = 510
ZROW = H * H
NW = 32
ROWS_PER_W = H // NW
NTAB = H * H + 1032
TAPS = [(t // 3, t % 3) for t in range(9)]


def _take16(v, lane):
    return lax.gather(
        v, lane[:, None],
        dimension_numbers=lax.GatherDimensionNumbers(
            offset_dims=(), collapsed_slice_dims=(0,), start_index_map=(0,)),
        slice_sizes=(1,),
        mode=lax.GatherScatterMode.PROMISE_IN_BOUNDS)


def _sc_warp(table, imap_pad):
    mesh = plsc.VectorSubcoreMesh(core_axis_name="c", subcore_axis_name="s")

    def body(tab_hbm, imap_hbm, out_hbm, mb_v, idx_v, g_v, acc_v, gs0, gs1):
        gsems = (gs0, gs1)
        wid = lax.axis_index("s") * 2 + lax.axis_index("c")
        y0 = wid * ROWS_PER_W

        pltpu.sync_copy(imap_hbm.at[pl.ds(y0, 24)], mb_v)

        zrow = jnp.full((16,), ZROW, jnp.int32)
        zero = jnp.zeros((16,), jnp.int32)
        hi = jnp.full((16,), HI, jnp.int32)
        iota = lax.iota(jnp.int32, 16)

        def dec_row(r, _):
            def dec_col(c, _):
                v = mb_v[r, pl.ds(c * 16, 16)]
                q = lax.div(v, hi)
                mb_v[r, pl.ds(c * 16, 16)] = jnp.where(v < zero, zrow, v + q + q)
                return 0
            return lax.fori_loop(0, 33, dec_col, 0)
        lax.fori_loop(0, 18, dec_row, 0)

        def build_and_fire(y, h, buf):
            x0 = h * 256
            for tap, (ki, kj) in enumerate(TAPS):
                row = y + 2 - ki
                sh = 2 - kj
                offv = jnp.full((16,), ki * H + kj, jnp.int32)
                lane = jnp.where(iota + sh < 16, iota + sh, iota + (sh - 16))
                hi_m = iota + sh >= 16

                def bld(c, _, row=row, sh=sh, offv=offv, lane=lane,
                        hi_m=hi_m, tap=tap, x0=x0):
                    base = x0 + c * 16
                    v0 = mb_v[row, pl.ds(base, 16)]
                    if sh == 0:
                        vb = v0
                    else:
                        v1 = mb_v[row, pl.ds(base + 16, 16)]
                        vb = jnp.where(
                            hi_m, _take16(v1, lane), _take16(v0, lane))
                    q = lax.shift_right_logical(c, 3)
                    idx_v[buf, tap * 2 + q, pl.ds((c & 7) * 16, 16)] = vb + offv
                    return 0
                lax.fori_loop(0, 16, bld, 0)
            for ch in range(18):
                pltpu.async_copy(
                    tab_hbm.at[idx_v.at[buf, ch]], g_v.at[buf, ch], gsems[buf])

        def do_stage(t, y, h, buf):
            @pl.when(t < 31)
            def _():
                t1 = t + 1
                build_and_fire(lax.shift_right_logical(t1, 1), t1 & 1, 1 - buf)

            for ch in range(18):
                pltpu.make_async_copy(
                    tab_hbm.at[idx_v.at[buf, ch]], g_v.at[buf, ch],
                    gsems[buf]).wait()

            x0 = h * 256

            def sum_body(p, _):
                q = lax.shift_right_logical(p, 7)
                l = p & 127
                v = [g_v[buf, tap * 2 + q, l, :] for tap in range(9)]
                s01 = v[0] + v[1]
                s23 = v[2] + v[3]
                s45 = v[4] + v[5]
                s67 = v[6] + v[7]
                acc_v[x0 + p, :] = ((s01 + s23) + (s45 + s67)) + v[8]
                return 0
            lax.fori_loop(0, 256, sum_body, 0)

            @pl.when(h == 1)
            def _():
                pltpu.sync_copy(acc_v, out_hbm.at[pl.ds((y0 + y) * H, H)])

        build_and_fire(jnp.int32(0), jnp.int32(0), 0)

        def pair(it, _):
            t = it * 2
            do_stage(t, lax.shift_right_logical(t, 1), t & 1, 0)
            do_stage(t + 1, lax.shift_right_logical(t + 1, 1), (t + 1) & 1, 1)
            return 0
        lax.fori_loop(0, 16, pair, 0)

    fn = pl.kernel(
        body,
        out_type=jax.ShapeDtypeStruct((H * H, 16), jnp.float32),
        mesh=mesh,
        scratch_types=[
            pltpu.VMEM((24, 528), jnp.int32),
            pltpu.VMEM((2, 18, 128), jnp.int32),
            pltpu.VMEM((2, 18, 128, 16), jnp.float32),
            pltpu.VMEM((H, 16), jnp.float32),
            pltpu.SemaphoreType.DMA,
            pltpu.SemaphoreType.DMA,
        ],
        compiler_params=pltpu.CompilerParams(use_tc_tiling_on_sc=False),
    )
    return fn(table, imap_pad)


@jax.jit
def kernel(lr, ref, index_map):
    del lr
    padded = jnp.pad(ref, ((0, 0), (0, 0), (1, 1), (1, 1)), mode='reflect')
    table = padded[0].transpose(1, 2, 0).reshape(H * H, 16)
    table = jnp.concatenate(
        [table, jnp.zeros((NTAB - H * H, 16), table.dtype)], axis=0)
    m2 = index_map.reshape(HI, HI).astype(jnp.int32)
    imp = jnp.full((520, 528), -1, jnp.int32)
    imp = lax.dynamic_update_slice(imp, m2, (2, 2))
    out = _sc_warp(table, imp)
    return out.reshape(H, H, 16).transpose(2, 0, 1)[None]


# speedup vs baseline: 2.2340x; 1.1309x over previous
import jax
import jax.numpy as jnp
from jax import lax
from jax.experimental import pallas as pl
from jax.experimental.pallas import tpu as pltpu
from jax.experimental.pallas import tpu_sc as plsc

H = 512
HI = 510
ZROW = H * H
NW = 32
ROWS_PER_W = H // NW
NTAB = H * H + 1032
TAPS = [(t // 3, t % 3) for t in range(9)]


def _take16(v, lane):
    return lax.gather(
        v, lane[:, None],
        dimension_numbers=lax.GatherDimensionNumbers(
            offset_dims=(), collapsed_slice_dims=(0,), start_index_map=(0,)),
        slice_sizes=(1,),
        mode=lax.GatherScatterMode.PROMISE_IN_BOUNDS)


def _sc_warp(table, imap_pad):
    mesh = plsc.VectorSubcoreMesh(core_axis_name="c", subcore_axis_name="s")

    def body(tab_hbm, imap_hbm, out_hbm, mb_v, idx_v, g_v, acc_t, gs0, gs1,
             os0, os1):
        gsems = (gs0, gs1)
        osems = (os0, os1)
        wid = lax.axis_index("s") * 2 + lax.axis_index("c")
        y0 = wid * ROWS_PER_W

        pltpu.sync_copy(imap_hbm.at[pl.ds(y0, 24)], mb_v)

        zrow = jnp.full((16,), ZROW, jnp.int32)
        zero = jnp.zeros((16,), jnp.int32)
        hi = jnp.full((16,), HI, jnp.int32)
        iota = lax.iota(jnp.int32, 16)

        def dec_row(r, _):
            def dec_col(c, _):
                v = mb_v[r, pl.ds(c * 16, 16)]
                q = lax.div(v, hi)
                mb_v[r, pl.ds(c * 16, 16)] = jnp.where(v < zero, zrow, v + q + q)
                return 0
            return lax.fori_loop(0, 33, dec_col, 0)
        lax.fori_loop(0, 18, dec_row, 0)

        def build_and_fire(y, h, buf):
            x0 = h * 256
            for tap, (ki, kj) in enumerate(TAPS):
                row = y + 2 - ki
                sh = 2 - kj
                offv = jnp.full((16,), ki * H + kj, jnp.int32)
                lane = jnp.where(iota + sh < 16, iota + sh, iota + (sh - 16))
                hi_m = iota + sh >= 16

                def bld(c, _, row=row, sh=sh, offv=offv, lane=lane,
                        hi_m=hi_m, tap=tap, x0=x0):
                    base = x0 + c * 16
                    v0 = mb_v[row, pl.ds(base, 16)]
                    if sh == 0:
                        vb = v0
                    else:
                        v1 = mb_v[row, pl.ds(base + 16, 16)]
                        vb = jnp.where(
                            hi_m, _take16(v1, lane), _take16(v0, lane))
                    q = lax.shift_right_logical(c, 3)
                    idx_v[buf, tap * 2 + q, pl.ds((c & 7) * 16, 16)] = vb + offv
                    return 0
                lax.fori_loop(0, 16, bld, 0)
            for ch in range(18):
                pltpu.async_copy(
                    tab_hbm.at[idx_v.at[buf, ch]], g_v.at[buf, ch], gsems[buf])

        def out_row_copies(y, parity):
            return [
                pltpu.make_async_copy(
                    acc_t.at[parity, c, pl.ds(0, H)],
                    out_hbm.at[pl.ds(c * (H * H) + (y0 + y) * H, H)],
                    osems[parity])
                for c in range(16)
            ]

        def do_stage(t, y, h, buf):
            parity = y & 1

            @pl.when(t < 31)
            def _():
                t1 = t + 1
                build_and_fire(lax.shift_right_logical(t1, 1), t1 & 1, 1 - buf)

            for pstat in range(2):
                @pl.when((h == 0) & (y >= 2) & (parity == pstat))
                def _(pstat=pstat):
                    for cp in out_row_copies(y - 2, pstat):
                        cp.wait()

            for ch in range(18):
                pltpu.make_async_copy(
                    tab_hbm.at[idx_v.at[buf, ch]], g_v.at[buf, ch],
                    gsems[buf]).wait()

            x0 = h * 256

            def sum_body(p, _):
                q = lax.shift_right_logical(p, 7)
                l = p & 127
                v = [g_v[buf, tap * 2 + q, l, :] for tap in range(9)]
                s01 = v[0] + v[1]
                s23 = v[2] + v[3]
                s45 = v[4] + v[5]
                s67 = v[6] + v[7]
                tot = ((s01 + s23) + (s45 + s67)) + v[8]
                plsc.store_scatter(
                    acc_t,
                    [jnp.full((16,), parity, jnp.int32), iota,
                     jnp.full((16,), x0 + p, jnp.int32)], tot)
                return 0
            lax.fori_loop(0, 256, sum_body, 0)

            for pstat in range(2):
                @pl.when((h == 1) & (parity == pstat))
                def _(pstat=pstat):
                    for cp in out_row_copies(y, pstat):
                        cp.start()

        build_and_fire(jnp.int32(0), jnp.int32(0), 0)

        def pair(it, _):
            t = it * 2
            do_stage(t, lax.shift_right_logical(t, 1), t & 1, 0)
            do_stage(t + 1, lax.shift_right_logical(t + 1, 1), (t + 1) & 1, 1)
            return 0
        lax.fori_loop(0, 16, pair, 0)

        for yy in (ROWS_PER_W - 2, ROWS_PER_W - 1):
            for cp in out_row_copies(jnp.int32(yy), yy & 1):
                cp.wait()

    fn = pl.kernel(
        body,
        out_type=jax.ShapeDtypeStruct((16 * H * H,), jnp.float32),
        mesh=mesh,
        scratch_types=[
            pltpu.VMEM((24, 528), jnp.int32),
            pltpu.VMEM((2, 18, 128), jnp.int32),
            pltpu.VMEM((2, 18, 128, 16), jnp.float32),
            pltpu.VMEM((2, 16, 513), jnp.float32),
            pltpu.SemaphoreType.DMA,
            pltpu.SemaphoreType.DMA,
            pltpu.SemaphoreType.DMA,
            pltpu.SemaphoreType.DMA,
        ],
        compiler_params=pltpu.CompilerParams(
            use_tc_tiling_on_sc=False, needs_layout_passes=False),
    )
    return fn(table, imap_pad)


@jax.jit
def kernel(lr, ref, index_map):
    del lr
    padded = jnp.pad(ref, ((0, 0), (0, 0), (1, 1), (1, 1)), mode='reflect')
    table = jnp.zeros((NTAB, 16), jnp.float32)
    table = lax.dynamic_update_slice(
        table, padded[0].transpose(1, 2, 0).reshape(H * H, 16), (0, 0))
    m2 = index_map.reshape(HI, HI).astype(jnp.int32)
    imp = jnp.full((520, 528), -1, jnp.int32)
    imp = lax.dynamic_update_slice(imp, m2, (2, 2))
    out = _sc_warp(table, imp)
    return out.reshape(16, H, H)[None]
